# baseline (speedup 1.0000x reference)
import functools

import jax
import jax.numpy as jnp
from jax import lax
from jax.experimental import pallas as pl
from jax.experimental.pallas import tpu as pltpu
from jax.experimental.pallas import tpu_sc as plsc

N = 1000
C = 151
NMS_THRESH = 0.3
BN_EPS = 1e-5
NP = 1024
NCHUNK = NP // 16
NCLS = C - 1

_info = plsc.get_sparse_core_info()
_NC = _info.num_cores
_NS = _info.num_subcores
_NW = _NC * _NS
_CPW = -(-NCLS // _NW)

_HIGH = jax.lax.Precision.HIGHEST


def _dot(a, b):
    return jnp.dot(a.astype(jnp.bfloat16), b.astype(jnp.bfloat16),
                   preferred_element_type=jnp.float32)


def _dense_body(fm, lg, pri, embw, w1t, b1, g, bta, mu, var, dw1t, dw2t, dw3t,
                db, d2_ref, pr_ref):
    l = lg[:]
    m = jnp.max(l, axis=1, keepdims=True)
    e = jnp.exp(l - m)
    p = e / jnp.sum(e, axis=1, keepdims=True)
    emb = _dot(p, embw[:])

    bp = pri[:]
    wh = bp[:, 2:4] - bp[:, 0:2] + 1.0
    ctr = bp[:, 0:2] + 0.5 * wh
    cs = jnp.concatenate([ctr, wh], axis=1)
    bn = (cs - mu[:]) / jnp.sqrt(var[:] + BN_EPS) * g[:] + bta[:]
    pos = jnp.maximum(_dot(bn, w1t[:]) + b1[:], 0.0)

    d2 = _dot(fm[:], dw1t[:]) + _dot(emb, dw2t[:]) + _dot(pos, dw3t[:]) + db[:]
    d2_ref[:] = d2

    m2 = jnp.max(d2, axis=1, keepdims=True)
    e2 = jnp.exp(d2 - m2)
    pr_ref[:] = e2 / jnp.sum(e2, axis=1, keepdims=True)


_dense_call = pl.pallas_call(
    _dense_body,
    out_shape=(jax.ShapeDtypeStruct((N, C), jnp.float32),
               jax.ShapeDtypeStruct((N, C), jnp.float32)),
)


def _nms_body(sc_hbm, x1_hbm, y1_hbm, x2_hbm, y2_hbm, out_hbm,
              sc_v, x1_v, y1_v, x2_v, y2_v, ar_v, kp_v):
    wid = lax.axis_index("c") * _NS + lax.axis_index("s")
    iota16 = lax.iota(jnp.int32, 16)
    big = jnp.int32(1 << 30)

    def xlane(v, s):
        return v.at[iota16 ^ s].get(mode="promise_in_bounds")

    def chunk_max(j, mv):
        return jnp.maximum(mv, sc_v[pl.ds(j * 16, 16)])

    def find_max():
        mv = lax.fori_loop(0, NCHUNK, chunk_max,
                           jnp.full((16,), -2.0, jnp.float32))
        for s in (8, 4, 2, 1):
            mv = jnp.maximum(mv, xlane(mv, s))

        def chunk_idx(j, iv):
            v = sc_v[pl.ds(j * 16, 16)]
            return jnp.minimum(iv, jnp.where(v == mv, iota16 + j * 16, big))

        iv = lax.fori_loop(0, NCHUNK, chunk_idx, jnp.full((16,), big, jnp.int32))
        for s in (8, 4, 2, 1):
            iv = jnp.minimum(iv, xlane(iv, s))
        return mv[0], iv[0]

    def per_class(k, _):
        c = k * _NW + wid

        @pl.when(c < NCLS)
        def _():
            pltpu.sync_copy(sc_hbm.at[c], sc_v)
            pltpu.sync_copy(x1_hbm.at[c], x1_v)
            pltpu.sync_copy(y1_hbm.at[c], y1_v)
            pltpu.sync_copy(x2_hbm.at[c], x2_v)
            pltpu.sync_copy(y2_hbm.at[c], y2_v)

            def init(j, _c):
                ds = pl.ds(j * 16, 16)
                ar_v[ds] = (x2_v[ds] - x1_v[ds] + 1.0) * (y2_v[ds] - y1_v[ds] + 1.0)
                kp_v[ds] = jnp.zeros((16,), jnp.float32)
                return 0

            lax.fori_loop(0, NCHUNK, init, 0)

            def body(st):
                _m, i = st
                ci = i // 16
                lane = i - ci * 16
                dsi = pl.ds(ci * 16, 16)
                lane_splat = jnp.zeros((16,), jnp.int32) + lane

                def pick(ref):
                    return ref[dsi].at[lane_splat].get(mode="promise_in_bounds")

                bx1 = pick(x1_v)
                by1 = pick(y1_v)
                bx2 = pick(x2_v)
                by2 = pick(y2_v)
                bar = pick(ar_v)
                kc = kp_v[dsi]
                kp_v[dsi] = jnp.where(iota16 == lane, 1.0, kc)

                def sup(j, _c):
                    ds = pl.ds(j * 16, 16)
                    x1 = x1_v[ds]
                    y1 = y1_v[ds]
                    x2 = x2_v[ds]
                    y2 = y2_v[ds]
                    xx1 = jnp.maximum(bx1, x1)
                    yy1 = jnp.maximum(by1, y1)
                    xx2 = jnp.minimum(bx2, x2)
                    yy2 = jnp.minimum(by2, y2)
                    w = jnp.maximum(xx2 - xx1 + 1.0, 0.0)
                    h = jnp.maximum(yy2 - yy1 + 1.0, 0.0)
                    inter = w * h
                    ar = (x2 - x1 + 1.0) * (y2 - y1 + 1.0)
                    iou = inter / ((bar + ar) - inter)
                    sv = sc_v[ds]
                    sc_v[ds] = jnp.where(iou > NMS_THRESH, -1.0, sv)
                    return 0

                lax.fori_loop(0, NCHUNK, sup, 0, unroll=2)
                return find_max()

            def step(_s, st):
                return lax.cond(st[0] > -0.5, body, lambda s: s, st)

            lax.fori_loop(0, NP, step, find_max())
            pltpu.sync_copy(kp_v, out_hbm.at[c])

        return 0

    lax.fori_loop(0, _CPW, per_class, 0)


_nms_call = pl.kernel(
    _nms_body,
    out_type=jax.ShapeDtypeStruct((NCLS, NP), jnp.float32),
    mesh=plsc.VectorSubcoreMesh(core_axis_name="c", subcore_axis_name="s"),
    scratch_types=[pltpu.VMEM((NP,), jnp.float32) for _ in range(7)],
)


def _argmax_body(pr, mk, out_ref):
    x = pr[:] * mk[:]
    iota = lax.broadcasted_iota(jnp.int32, (N, C), 1)
    valid = iota >= 1
    xv = jnp.where(valid, x, -1.0)
    m = jnp.max(xv, axis=1, keepdims=True)
    idx = jnp.min(jnp.where((xv == m) & valid, iota, jnp.int32(1 << 30)),
                  axis=1, keepdims=True)
    out_ref[:] = idx


_argmax_call = pl.pallas_call(
    _argmax_body,
    out_shape=jax.ShapeDtypeStruct((N, 1), jnp.int32),
)


def kernel(obj_fmaps, obj_logits, im_inds, box_priors, boxes_per_cls,
           obj_embed_weight, bn_gamma, bn_beta, bn_mean, bn_var,
           lin1_W, lin1_b, dec_W, dec_b):
    d2, probs = _dense_call(
        obj_fmaps, obj_logits, box_priors, obj_embed_weight,
        lin1_W.T, lin1_b.reshape(1, -1),
        bn_gamma.reshape(1, -1), bn_beta.reshape(1, -1),
        bn_mean.reshape(1, -1), bn_var.reshape(1, -1),
        dec_W[:, :4096].T, dec_W[:, 4096:4116].T, dec_W[:, 4116:].T,
        dec_b.reshape(1, -1))

    scoresT = jnp.pad(probs[:, 1:].T, ((0, 0), (0, NP - N)),
                      constant_values=-1.0)
    bt = jnp.swapaxes(boxes_per_cls, 0, 1)[1:]
    pads = ((0, 0), (0, NP - N))
    x1t = jnp.pad(bt[..., 0], pads)
    y1t = jnp.pad(bt[..., 1], pads)
    x2t = jnp.pad(bt[..., 2], pads)
    y2t = jnp.pad(bt[..., 3], pads)

    mask = _nms_call(scoresT, x1t, y1t, x2t, y2t)

    mask_full = jnp.concatenate(
        [jnp.zeros((N, 1), jnp.float32), mask[:, :N].T], axis=1)
    preds = _argmax_call(probs, mask_full)
    return d2, preds.reshape(N)


# speedup vs baseline: 1.9611x; 1.9611x over previous
import functools

import jax
import jax.numpy as jnp
from jax import lax
from jax.experimental import pallas as pl
from jax.experimental.pallas import tpu as pltpu
from jax.experimental.pallas import tpu_sc as plsc

N = 1000
C = 151
NMS_THRESH = 0.3
BN_EPS = 1e-5
NP = 1024
NCHUNK = NP // 16
NCLS = C - 1

_info = plsc.get_sparse_core_info()
_NC = _info.num_cores
_NS = _info.num_subcores
_NW = _NC * _NS
_CPW = -(-NCLS // _NW)

_HIGH = jax.lax.Precision.HIGHEST


def _dot(a, b):
    return jnp.dot(a.astype(jnp.bfloat16), b.astype(jnp.bfloat16),
                   preferred_element_type=jnp.float32)


def _dense_body(fm, lg, pri, embw, w1t, b1, g, bta, mu, var, dw1t, dw2t, dw3t,
                db, d2_ref, pr_ref):
    l = lg[:]
    m = jnp.max(l, axis=1, keepdims=True)
    e = jnp.exp(l - m)
    p = e / jnp.sum(e, axis=1, keepdims=True)
    emb = _dot(p, embw[:])

    bp = pri[:]
    wh = bp[:, 2:4] - bp[:, 0:2] + 1.0
    ctr = bp[:, 0:2] + 0.5 * wh
    cs = jnp.concatenate([ctr, wh], axis=1)
    bn = (cs - mu[:]) / jnp.sqrt(var[:] + BN_EPS) * g[:] + bta[:]
    pos = jnp.maximum(_dot(bn, w1t[:]) + b1[:], 0.0)

    d2 = _dot(fm[:], dw1t[:]) + _dot(emb, dw2t[:]) + _dot(pos, dw3t[:]) + db[:]
    d2_ref[:] = d2

    m2 = jnp.max(d2, axis=1, keepdims=True)
    e2 = jnp.exp(d2 - m2)
    pr_ref[:] = e2 / jnp.sum(e2, axis=1, keepdims=True)


_dense_call = pl.pallas_call(
    _dense_body,
    out_shape=(jax.ShapeDtypeStruct((N, C), jnp.float32),
               jax.ShapeDtypeStruct((N, C), jnp.float32)),
)


def _nms_body(sc_hbm, x1_hbm, y1_hbm, x2_hbm, y2_hbm, out_hbm,
              sc_v, x1_v, y1_v, x2_v, y2_v, ar_v, kp_v):
    wid = lax.axis_index("c") * _NS + lax.axis_index("s")
    iota16 = lax.iota(jnp.int32, 16)
    big = jnp.int32(1 << 30)

    def xlane(v, s):
        return v.at[iota16 ^ s].get(mode="promise_in_bounds")

    def chunk_max(j, mv):
        return jnp.maximum(mv, sc_v[pl.ds(j * 16, 16)])

    def find_max():
        mv = lax.fori_loop(0, NCHUNK, chunk_max,
                           jnp.full((16,), -2.0, jnp.float32))
        for s in (8, 4, 2, 1):
            mv = jnp.maximum(mv, xlane(mv, s))

        def chunk_idx(j, iv):
            v = sc_v[pl.ds(j * 16, 16)]
            return jnp.minimum(iv, jnp.where(v == mv, iota16 + j * 16, big))

        iv = lax.fori_loop(0, NCHUNK, chunk_idx, jnp.full((16,), big, jnp.int32))
        for s in (8, 4, 2, 1):
            iv = jnp.minimum(iv, xlane(iv, s))
        return mv[0], iv[0]

    def per_class(k, _):
        c = k * _NW + wid

        @pl.when(c < NCLS)
        def _():
            pltpu.sync_copy(sc_hbm.at[c], sc_v)
            pltpu.sync_copy(x1_hbm.at[c], x1_v)
            pltpu.sync_copy(y1_hbm.at[c], y1_v)
            pltpu.sync_copy(x2_hbm.at[c], x2_v)
            pltpu.sync_copy(y2_hbm.at[c], y2_v)

            def init(j, _c):
                ds = pl.ds(j * 16, 16)
                ar_v[ds] = (x2_v[ds] - x1_v[ds] + 1.0) * (y2_v[ds] - y1_v[ds] + 1.0)
                kp_v[ds] = jnp.zeros((16,), jnp.float32)
                return 0

            lax.fori_loop(0, NCHUNK, init, 0)

            def body(st):
                _m, i = st
                ci = i // 16
                lane = i - ci * 16
                dsi = pl.ds(ci * 16, 16)
                lane_splat = jnp.zeros((16,), jnp.int32) + lane

                def pick(ref):
                    return ref[dsi].at[lane_splat].get(mode="promise_in_bounds")

                bx1 = pick(x1_v)
                by1 = pick(y1_v)
                bx2 = pick(x2_v)
                by2 = pick(y2_v)
                bar = pick(ar_v)
                kc = kp_v[dsi]
                kp_v[dsi] = jnp.where(iota16 == lane, 1.0, kc)

                def sup(j, _c):
                    ds = pl.ds(j * 16, 16)
                    x1 = x1_v[ds]
                    y1 = y1_v[ds]
                    x2 = x2_v[ds]
                    y2 = y2_v[ds]
                    xx1 = jnp.maximum(bx1, x1)
                    yy1 = jnp.maximum(by1, y1)
                    xx2 = jnp.minimum(bx2, x2)
                    yy2 = jnp.minimum(by2, y2)
                    w = jnp.maximum(xx2 - xx1 + 1.0, 0.0)
                    h = jnp.maximum(yy2 - yy1 + 1.0, 0.0)
                    inter = w * h
                    ar = (x2 - x1 + 1.0) * (y2 - y1 + 1.0)
                    iou = inter / ((bar + ar) - inter)
                    sv = sc_v[ds]
                    sc_v[ds] = jnp.where(iou > NMS_THRESH, -1.0, sv)
                    return 0

                lax.fori_loop(0, NCHUNK, sup, 0)
                return find_max()

            def step(_s, st):
                return lax.cond(st[0] > -0.5, body, lambda s: s, st)

            lax.fori_loop(0, NP, step, find_max())
            pltpu.sync_copy(kp_v, out_hbm.at[c])

        return 0

    lax.fori_loop(0, _CPW, per_class, 0)


_nms_call = pl.kernel(
    _nms_body,
    out_type=jax.ShapeDtypeStruct((NCLS, NP), jnp.float32),
    mesh=plsc.VectorSubcoreMesh(core_axis_name="c", subcore_axis_name="s"),
    scratch_types=[pltpu.VMEM((NP,), jnp.float32) for _ in range(7)],
)


def _argmax_body(pr, mk, out_ref):
    x = pr[:] * mk[:]
    iota = lax.broadcasted_iota(jnp.int32, (N, C), 1)
    valid = iota >= 1
    xv = jnp.where(valid, x, -1.0)
    m = jnp.max(xv, axis=1, keepdims=True)
    idx = jnp.min(jnp.where((xv == m) & valid, iota, jnp.int32(1 << 30)),
                  axis=1, keepdims=True)
    out_ref[:] = idx


_argmax_call = pl.pallas_call(
    _argmax_body,
    out_shape=jax.ShapeDtypeStruct((N, 1), jnp.int32),
)


def kernel(obj_fmaps, obj_logits, im_inds, box_priors, boxes_per_cls,
           obj_embed_weight, bn_gamma, bn_beta, bn_mean, bn_var,
           lin1_W, lin1_b, dec_W, dec_b):
    d2, probs = _dense_call(
        obj_fmaps, obj_logits, box_priors, obj_embed_weight,
        lin1_W.T, lin1_b.reshape(1, -1),
        bn_gamma.reshape(1, -1), bn_beta.reshape(1, -1),
        bn_mean.reshape(1, -1), bn_var.reshape(1, -1),
        dec_W[:, :4096].T, dec_W[:, 4096:4116].T, dec_W[:, 4116:].T,
        dec_b.reshape(1, -1))

    scoresT = jnp.pad(probs[:, 1:].T, ((0, 0), (0, NP - N)),
                      constant_values=-1.0)
    bt = jnp.swapaxes(boxes_per_cls, 0, 1)[1:]
    pads = ((0, 0), (0, NP - N))
    x1t = jnp.pad(bt[..., 0], pads)
    y1t = jnp.pad(bt[..., 1], pads)
    x2t = jnp.pad(bt[..., 2], pads)
    y2t = jnp.pad(bt[..., 3], pads)

    mask = _nms_call(scoresT, x1t, y1t, x2t, y2t)

    mask_full = jnp.concatenate(
        [jnp.zeros((N, 1), jnp.float32), mask[:, :N].T], axis=1)
    preds = _argmax_call(probs, mask_full)
    return d2, preds.reshape(N)


# speedup vs baseline: 2.8537x; 1.4552x over previous
import functools

import jax
import jax.numpy as jnp
from jax import lax
from jax.experimental import pallas as pl
from jax.experimental.pallas import tpu as pltpu
from jax.experimental.pallas import tpu_sc as plsc

N = 1000
C = 151
NMS_THRESH = 0.3
BN_EPS = 1e-5
NP = 1024
NCHUNK = NP // 16
NCLS = C - 1

_info = plsc.get_sparse_core_info()
_NC = _info.num_cores
_NS = _info.num_subcores
_NW = _NC * _NS
_CPW = -(-NCLS // _NW)

_HIGH = jax.lax.Precision.HIGHEST


def _dot(a, b):
    return jnp.dot(a.astype(jnp.bfloat16), b.astype(jnp.bfloat16),
                   preferred_element_type=jnp.float32)


def _dense_body(fm, lg, pri, embw, w1t, b1, g, bta, mu, var, dw1t, dw2t, dw3t,
                db, d2_ref, pr_ref):
    l = lg[:]
    m = jnp.max(l, axis=1, keepdims=True)
    e = jnp.exp(l - m)
    p = e / jnp.sum(e, axis=1, keepdims=True)
    emb = _dot(p, embw[:])

    bp = pri[:]
    wh = bp[:, 2:4] - bp[:, 0:2] + 1.0
    ctr = bp[:, 0:2] + 0.5 * wh
    cs = jnp.concatenate([ctr, wh], axis=1)
    bn = (cs - mu[:]) / jnp.sqrt(var[:] + BN_EPS) * g[:] + bta[:]
    pos = jnp.maximum(_dot(bn, w1t[:]) + b1[:], 0.0)

    d2 = _dot(fm[:], dw1t[:]) + _dot(emb, dw2t[:]) + _dot(pos, dw3t[:]) + db[:]
    d2_ref[:] = d2

    m2 = jnp.max(d2, axis=1, keepdims=True)
    e2 = jnp.exp(d2 - m2)
    pr_ref[:] = e2 / jnp.sum(e2, axis=1, keepdims=True)


_dense_call = pl.pallas_call(
    _dense_body,
    out_shape=(jax.ShapeDtypeStruct((N, C), jnp.float32),
               jax.ShapeDtypeStruct((N, C), jnp.float32)),
)


def _nms_body(sc_hbm, x1_hbm, y1_hbm, x2_hbm, y2_hbm, out_hbm,
              sc_v, x1_v, y1_v, x2_v, y2_v, ar_v, kp_v):
    wid = lax.axis_index("c") * _NS + lax.axis_index("s")
    iota16 = lax.iota(jnp.int32, 16)
    big = jnp.int32(1 << 30)

    def xlane(v, s):
        return v.at[iota16 ^ s].get(mode="promise_in_bounds")

    def chunk_max(j, mv):
        return jnp.maximum(mv, sc_v[pl.ds(j * 16, 16)])

    def find_max():
        mv = lax.fori_loop(0, NCHUNK, chunk_max,
                           jnp.full((16,), -2.0, jnp.float32))
        for s in (8, 4, 2, 1):
            mv = jnp.maximum(mv, xlane(mv, s))

        def chunk_idx(j, iv):
            v = sc_v[pl.ds(j * 16, 16)]
            return jnp.minimum(iv, jnp.where(v == mv, iota16 + j * 16, big))

        iv = lax.fori_loop(0, NCHUNK, chunk_idx, jnp.full((16,), big, jnp.int32))
        for s in (8, 4, 2, 1):
            iv = jnp.minimum(iv, xlane(iv, s))
        return mv[0], iv[0]

    def per_class(k, _):
        c = k * _NW + wid

        @pl.when(c < NCLS)
        def _():
            pltpu.sync_copy(sc_hbm.at[c], sc_v)
            pltpu.sync_copy(x1_hbm.at[c], x1_v)
            pltpu.sync_copy(y1_hbm.at[c], y1_v)
            pltpu.sync_copy(x2_hbm.at[c], x2_v)
            pltpu.sync_copy(y2_hbm.at[c], y2_v)

            def init(j, _c):
                ds = pl.ds(j * 16, 16)
                ar_v[ds] = (x2_v[ds] - x1_v[ds] + 1.0) * (y2_v[ds] - y1_v[ds] + 1.0)
                kp_v[ds] = jnp.zeros((16,), jnp.float32)
                return 0

            lax.fori_loop(0, NCHUNK, init, 0)

            def body(st):
                _m, i = st
                ci = i // 16
                lane = i - ci * 16
                dsi = pl.ds(ci * 16, 16)
                lane_splat = jnp.zeros((16,), jnp.int32) + lane

                def pick(ref):
                    return ref[dsi].at[lane_splat].get(mode="promise_in_bounds")

                bx1 = pick(x1_v)
                by1 = pick(y1_v)
                bx2 = pick(x2_v)
                by2 = pick(y2_v)
                bar = pick(ar_v)
                kc = kp_v[dsi]
                kp_v[dsi] = jnp.where(iota16 == lane, 1.0, kc)

                def sup(j, mv):
                    ds = pl.ds(j * 16, 16)
                    xx1 = jnp.maximum(bx1, x1_v[ds])
                    yy1 = jnp.maximum(by1, y1_v[ds])
                    xx2 = jnp.minimum(bx2, x2_v[ds])
                    yy2 = jnp.minimum(by2, y2_v[ds])
                    w = jnp.maximum(xx2 - xx1 + 1.0, 0.0)
                    h = jnp.maximum(yy2 - yy1 + 1.0, 0.0)
                    inter = w * h
                    iou = inter / ((bar + ar_v[ds]) - inter)
                    sv = sc_v[ds]
                    nsv = jnp.where(iou > NMS_THRESH, -1.0, sv)
                    sc_v[ds] = nsv
                    return jnp.maximum(mv, nsv)

                mv = lax.fori_loop(0, NCHUNK, sup,
                                   jnp.full((16,), -2.0, jnp.float32))
                for s in (8, 4, 2, 1):
                    mv = jnp.maximum(mv, xlane(mv, s))

                def chunk_idx(j, iv):
                    v = sc_v[pl.ds(j * 16, 16)]
                    return jnp.minimum(
                        iv, jnp.where(v == mv, iota16 + j * 16, big))

                iv = lax.fori_loop(0, NCHUNK, chunk_idx,
                                   jnp.full((16,), big, jnp.int32))
                for s in (8, 4, 2, 1):
                    iv = jnp.minimum(iv, xlane(iv, s))
                return mv[0], iv[0]

            def step(_s, st):
                return lax.cond(st[0] > -0.5, body, lambda s: s, st)

            lax.fori_loop(0, NP, step, find_max())
            pltpu.sync_copy(kp_v, out_hbm.at[c])

        return 0

    lax.fori_loop(0, _CPW, per_class, 0)


_nms_call = pl.kernel(
    _nms_body,
    out_type=jax.ShapeDtypeStruct((NCLS, NP), jnp.float32),
    mesh=plsc.VectorSubcoreMesh(core_axis_name="c", subcore_axis_name="s"),
    scratch_types=[pltpu.VMEM((NP,), jnp.float32) for _ in range(7)],
)


def _argmax_body(pr, mk, out_ref):
    x = pr[:] * mk[:]
    iota = lax.broadcasted_iota(jnp.int32, (N, C), 1)
    valid = iota >= 1
    xv = jnp.where(valid, x, -1.0)
    m = jnp.max(xv, axis=1, keepdims=True)
    idx = jnp.min(jnp.where((xv == m) & valid, iota, jnp.int32(1 << 30)),
                  axis=1, keepdims=True)
    out_ref[:] = idx


_argmax_call = pl.pallas_call(
    _argmax_body,
    out_shape=jax.ShapeDtypeStruct((N, 1), jnp.int32),
)


def kernel(obj_fmaps, obj_logits, im_inds, box_priors, boxes_per_cls,
           obj_embed_weight, bn_gamma, bn_beta, bn_mean, bn_var,
           lin1_W, lin1_b, dec_W, dec_b):
    d2, probs = _dense_call(
        obj_fmaps, obj_logits, box_priors, obj_embed_weight,
        lin1_W.T, lin1_b.reshape(1, -1),
        bn_gamma.reshape(1, -1), bn_beta.reshape(1, -1),
        bn_mean.reshape(1, -1), bn_var.reshape(1, -1),
        dec_W[:, :4096].T, dec_W[:, 4096:4116].T, dec_W[:, 4116:].T,
        dec_b.reshape(1, -1))

    scoresT = jnp.pad(probs[:, 1:].T, ((0, 0), (0, NP - N)),
                      constant_values=-1.0)
    bt = jnp.swapaxes(boxes_per_cls, 0, 1)[1:]
    pads = ((0, 0), (0, NP - N))
    x1t = jnp.pad(bt[..., 0], pads)
    y1t = jnp.pad(bt[..., 1], pads)
    x2t = jnp.pad(bt[..., 2], pads)
    y2t = jnp.pad(bt[..., 3], pads)

    mask = _nms_call(scoresT, x1t, y1t, x2t, y2t)

    mask_full = jnp.concatenate(
        [jnp.zeros((N, 1), jnp.float32), mask[:, :N].T], axis=1)
    preds = _argmax_call(probs, mask_full)
    return d2, preds.reshape(N)
